# baseline (speedup 1.0000x reference)
import functools

import jax
import jax.numpy as jnp
from jax import lax
from jax.experimental import pallas as pl
from jax.experimental.pallas import tpu as pltpu
from jax.experimental.pallas import tpu_sc as plsc

_WLOG = 11
_W = 1 << _WLOG
_NBUF = 2
_CAP = 1024
_XCH = 64


def _fused_call(idx, dbt, tail_t, n_full):
    B, = idx.shape
    D, V = dbt.shape
    Dt, T = tail_t.shape
    info = plsc.get_sparse_core_info()
    NC, NS, L = info.num_cores, info.num_subcores, info.num_lanes
    NW = NC * NS
    max_j = -(-n_full // NW)
    n_scan = B // L

    mesh = plsc.VectorSubcoreMesh(core_axis_name="c", subcore_axis_name="s")

    @functools.partial(
        pl.kernel,
        mesh=mesh,
        compiler_params=pltpu.CompilerParams(needs_layout_passes=False),
        out_type=jax.ShapeDtypeStruct((B, 128), jnp.float32),
        scratch_types=[
            pltpu.VMEM((B // 2,), jnp.int32),
            pltpu.VMEM((_CAP + L,), jnp.int32),
            pltpu.VMEM((_CAP + L,), jnp.int32),
            pltpu.VMEM((_NBUF, D, _W), jnp.float32),
            pltpu.VMEM((_CAP // 8, 128), jnp.float32),
            pltpu.VMEM((Dt, T), jnp.float32),
            pltpu.VMEM((2, _XCH, 128), jnp.float32),
            pltpu.SemaphoreType.DMA((_NBUF,)),
            pltpu.SemaphoreType.DMA((2,)),
            pltpu.SemaphoreType.DMA,
        ],
    )
    def k(idx_hbm, dbt_hbm, tail_hbm, out_hbm, idx_v, lidx, lpos, wbuf,
          ostage, tail_v, xbuf, wsem, xsem, sem):
        wid = lax.axis_index("s") * NC + lax.axis_index("c")

        for u in range(_NBUF):
            pltpu.async_copy(
                dbt_hbm.at[:, pl.ds((wid + u * NW) * _W, _W)],
                wbuf.at[u], wsem.at[u])
        pltpu.async_copy(tail_hbm, tail_v, sem).wait()

        def init_pos(t, _):
            lpos[pl.ds(t * L, L)] = jnp.full((L,), -1, jnp.int32)
            return _

        lax.fori_loop(0, (_CAP + L) // L, init_pos, 0)

        cnt = jnp.zeros((L,), jnp.int32)
        for half in range(2):
            pltpu.async_copy(
                idx_hbm.at[pl.ds(half * (B // 2), B // 2)], idx_v, sem
            ).wait()

            def scan(t, c, half=half):
                idxs = idx_v[pl.ds(t * L, L)]
                k_of = lax.shift_right_logical(idxs, _WLOG)
                m = (k_of & (NW - 1)) == wid
                mi = m.astype(jnp.int32)
                slot = c + plsc.cumsum(mi) - 1
                plsc.store_scatter(lidx, [slot], idxs, mask=m)
                b_vec = (lax.iota(jnp.int32, L) + (half * (B // 2) + t * L))
                plsc.store_scatter(lpos, [slot], b_vec, mask=m)
                return jnp.minimum(
                    c + plsc.all_reduce_population_count(m), _CAP)

            cnt = lax.fori_loop(0, n_scan // 2, scan, cnt)
        n_hit = jnp.max(cnt)
        n_vreg = lax.shift_right_logical(n_hit + (L - 1), 4)

        def pair(p, _):
            kw0 = wid + (2 * p) * NW
            kw1 = kw0 + NW

            @pl.when(kw0 < n_full)
            def _process():
                pltpu.make_async_copy(
                    dbt_hbm.at[:, pl.ds(0, _W)], wbuf.at[0], wsem.at[0]
                ).wait()

                @pl.when(kw1 < n_full)
                def _wait1():
                    pltpu.make_async_copy(
                        dbt_hbm.at[:, pl.ds(0, _W)], wbuf.at[0], wsem.at[1]
                    ).wait()

                have1 = kw1 < n_full

                def rescan(t, _):
                    idxs = lidx[pl.ds(t * L, L)]
                    in_rng = (lax.iota(jnp.int32, L) + t * L) < n_hit
                    k_of = lax.shift_right_logical(idxs, _WLOG)
                    m1 = (k_of == kw1) & in_rng & have1
                    m = ((k_of == kw0) & in_rng) | m1

                    @pl.when(jnp.any(m))
                    def _extract():
                        u_vec = m1.astype(jnp.int32)
                        off = idxs & (_W - 1)
                        p_vec = lax.iota(jnp.int32, L) + t * L
                        prow = lax.shift_right_logical(p_vec, 3)
                        pcol = (p_vec & 7) << 4
                        for d in range(D):
                            vals = plsc.load_gather(
                                wbuf, [u_vec, jnp.full((L,), d, jnp.int32),
                                       off],
                                mask=m)
                            plsc.store_scatter(
                                ostage, [prow, pcol + d], vals, mask=m)

                    return _

                lax.fori_loop(0, n_vreg, rescan, 0)

                for u, kn in ((0, kw0 + 2 * NW), (1, kw1 + 2 * NW)):

                    @pl.when(kn < n_full)
                    def _refill(u=u, kn=kn):
                        pltpu.async_copy(
                            dbt_hbm.at[:, pl.ds(kn * _W, _W)],
                            wbuf.at[u], wsem.at[u])

            return _

        lax.fori_loop(0, -(-max_j // 2), pair, 0)

        tail0 = n_full * _W

        def tail_rescan(t, _):
            idxs = lidx[pl.ds(t * L, L)]
            in_rng = (lax.iota(jnp.int32, L) + t * L) < n_hit
            m = (lax.shift_right_logical(idxs, _WLOG) == n_full) & in_rng

            @pl.when(jnp.any(m))
            def _extract():
                off = idxs - tail0
                p_vec = lax.iota(jnp.int32, L) + t * L
                prow = lax.shift_right_logical(p_vec, 3)
                pcol = (p_vec & 7) << 4
                for d in range(D):
                    vals = plsc.load_gather(
                        tail_v, [jnp.full((L,), d, jnp.int32), off], mask=m)
                    plsc.store_scatter(ostage, [prow, pcol + d], vals, mask=m)

            return _

        lax.fori_loop(0, n_vreg, tail_rescan, 0)

        n_chunk = _CAP // _XCH
        pending = {}
        for c in range(n_chunk):
            u = c & 1
            if u in pending:
                pending.pop(u).wait()

            def expand(r, _, c=c, u=u):
                g = c * _XCH + r
                xbuf[u, r, pl.ds(0, D)] = ostage[
                    lax.shift_right_logical(g, 3), pl.ds((g & 7) * D, D)]
                return _

            lax.fori_loop(0, _XCH, expand, 0)
            pending[u] = pltpu.async_copy(
                xbuf.at[u],
                out_hbm.at[plsc.Indices(lpos.at[pl.ds(c * _XCH, _XCH)],
                                        ignored_value=-1)],
                xsem.at[u],
            )
        for cp in pending.values():
            cp.wait()

    return k(idx, dbt, tail_t)


def kernel(idx, db):
    V, D = db.shape
    n_full = V // _W
    tail0 = n_full * _W
    dbt = db.T
    padded = _fused_call(idx.astype(jnp.int32), dbt, dbt[:, tail0:], n_full)
    return padded[:, :D]


# speedup vs baseline: 1.1196x; 1.1196x over previous
import functools

import jax
import jax.numpy as jnp
from jax import lax
from jax.experimental import pallas as pl
from jax.experimental.pallas import tpu as pltpu
from jax.experimental.pallas import tpu_sc as plsc

_WLOG = 11
_W = 1 << _WLOG
_NBUF = 2
_CAP = 1024
_XCH = 64


def _fused_call(idx, dbt, tail_t, n_full):
    B, = idx.shape
    D, V = dbt.shape
    Dt, T = tail_t.shape
    info = plsc.get_sparse_core_info()
    NC, NS, L = info.num_cores, info.num_subcores, info.num_lanes
    NW = NC * NS
    max_j = -(-n_full // NW)
    n_scan = B // L

    mesh = plsc.VectorSubcoreMesh(core_axis_name="c", subcore_axis_name="s")

    @functools.partial(
        pl.kernel,
        mesh=mesh,
        compiler_params=pltpu.CompilerParams(needs_layout_passes=False),
        out_type=jax.ShapeDtypeStruct((B, 128), jnp.float32),
        scratch_types=[
            pltpu.VMEM((B // 2,), jnp.int32),
            pltpu.VMEM((_CAP + L,), jnp.int32),
            pltpu.VMEM((_CAP + L,), jnp.int32),
            pltpu.VMEM((_NBUF, D, _W), jnp.float32),
            pltpu.VMEM((_CAP // 8, 128), jnp.float32),
            pltpu.VMEM((Dt, T), jnp.float32),
            pltpu.VMEM((2, _XCH, 128), jnp.float32),
            pltpu.SemaphoreType.DMA((_NBUF,)),
            pltpu.SemaphoreType.DMA((2,)),
            pltpu.SemaphoreType.DMA,
        ],
    )
    def k(idx_hbm, dbt_hbm, tail_hbm, out_hbm, idx_v, lidx, lpos, wbuf,
          ostage, tail_v, xbuf, wsem, xsem, sem):
        wid = lax.axis_index("s") * NC + lax.axis_index("c")

        for u in range(_NBUF):
            pltpu.async_copy(
                dbt_hbm.at[:, pl.ds((wid + u * NW) * _W, _W)],
                wbuf.at[u], wsem.at[u])
        pltpu.async_copy(tail_hbm, tail_v, sem).wait()

        def init_pos(t, _):
            lpos[pl.ds(t * L, L)] = jnp.full((L,), -1, jnp.int32)
            return _

        lax.fori_loop(0, (_CAP + L) // L, init_pos, 0)

        cnt = jnp.zeros((L,), jnp.int32)
        for half in range(2):
            pltpu.async_copy(
                idx_hbm.at[pl.ds(half * (B // 2), B // 2)], idx_v, sem
            ).wait()

            def scan(t, c, half=half):
                idxs = idx_v[pl.ds(t * L, L)]
                k_of = lax.shift_right_logical(idxs, _WLOG)
                m = (k_of & (NW - 1)) == wid
                mi = m.astype(jnp.int32)
                slot = c + plsc.cumsum(mi) - 1
                plsc.store_scatter(lidx, [slot], idxs, mask=m)
                b_vec = (lax.iota(jnp.int32, L) + (half * (B // 2) + t * L))
                plsc.store_scatter(lpos, [slot], b_vec, mask=m)
                return jnp.minimum(
                    c + plsc.all_reduce_population_count(m), _CAP)

            cnt = lax.fori_loop(0, n_scan // 2, scan, cnt)
        n_hit = jnp.max(cnt)
        n_vreg = lax.shift_right_logical(n_hit + (L - 1), 4)

        def do_window(j, par):
            kw = wid + j * NW

            @pl.when(kw < n_full)
            def _process():
                pltpu.make_async_copy(
                    dbt_hbm.at[:, pl.ds(0, _W)], wbuf.at[0], wsem.at[par]
                ).wait()

                def rescan(t, _):
                    idxs = lidx[pl.ds(t * L, L)]
                    in_rng = (lax.iota(jnp.int32, L) + t * L) < n_hit
                    m = ((lax.shift_right_logical(idxs, _WLOG) == kw)
                         & in_rng)

                    @pl.when(jnp.any(m))
                    def _extract():
                        off = idxs & (_W - 1)
                        p_vec = lax.iota(jnp.int32, L) + t * L
                        prow = lax.shift_right_logical(p_vec, 3)
                        pcol = (p_vec & 7) << 4
                        for d in range(D):
                            vals = plsc.load_gather(
                                wbuf, [jnp.full((L,), par, jnp.int32),
                                       jnp.full((L,), d, jnp.int32), off],
                                mask=m)
                            plsc.store_scatter(
                                ostage, [prow, pcol + d], vals, mask=m)

                    return _

                lax.fori_loop(0, n_vreg, rescan, 0)

                kn = kw + _NBUF * NW

                @pl.when(kn < n_full)
                def _refill():
                    pltpu.async_copy(
                        dbt_hbm.at[:, pl.ds(kn * _W, _W)],
                        wbuf.at[par], wsem.at[par])

        def window_group(g, _):
            for u in range(_NBUF):
                do_window(g * _NBUF + u, u)
            return _

        lax.fori_loop(0, -(-max_j // _NBUF), window_group, 0)

        tail0 = n_full * _W

        def tail_rescan(t, _):
            idxs = lidx[pl.ds(t * L, L)]
            in_rng = (lax.iota(jnp.int32, L) + t * L) < n_hit
            m = (lax.shift_right_logical(idxs, _WLOG) == n_full) & in_rng

            @pl.when(jnp.any(m))
            def _extract():
                off = idxs - tail0
                p_vec = lax.iota(jnp.int32, L) + t * L
                prow = lax.shift_right_logical(p_vec, 3)
                pcol = (p_vec & 7) << 4
                for d in range(D):
                    vals = plsc.load_gather(
                        tail_v, [jnp.full((L,), d, jnp.int32), off], mask=m)
                    plsc.store_scatter(ostage, [prow, pcol + d], vals, mask=m)

            return _

        lax.fori_loop(0, n_vreg, tail_rescan, 0)

        n_chunk = _CAP // _XCH
        pending = {}
        for c in range(n_chunk):
            u = c & 1
            if u in pending:
                pending.pop(u).wait()

            def expand(r, _, c=c, u=u):
                g = c * _XCH + r
                xbuf[u, r, pl.ds(0, D)] = ostage[
                    lax.shift_right_logical(g, 3), pl.ds((g & 7) * D, D)]
                return _

            lax.fori_loop(0, _XCH, expand, 0)
            pending[u] = pltpu.async_copy(
                xbuf.at[u],
                out_hbm.at[plsc.Indices(lpos.at[pl.ds(c * _XCH, _XCH)],
                                        ignored_value=-1)],
                xsem.at[u],
            )
        for cp in pending.values():
            cp.wait()

    return k(idx, dbt, tail_t)


def kernel(idx, db):
    V, D = db.shape
    n_full = V // _W
    tail0 = n_full * _W
    dbt = db.T
    padded = _fused_call(idx.astype(jnp.int32), dbt, dbt[:, tail0:], n_full)
    return padded[:, :D]
